# baseline (speedup 1.0000x reference)
import functools

import jax
import jax.numpy as jnp
from jax import lax
from jax.experimental import pallas as pl
from jax.experimental.pallas import tpu as pltpu
from jax.experimental.pallas import tpu_sc as plsc

_NUM_WORKERS = 32


@functools.partial(jax.jit, static_argnames=("wg",))
def _sc_gather(table, idx, wg=1000):
    m = idx.shape[0]
    d = table.shape[1]
    per_w = m // _NUM_WORKERS
    mesh = plsc.VectorSubcoreMesh(core_axis_name="c", subcore_axis_name="s")

    @functools.partial(
        pl.kernel, mesh=mesh,
        out_type=jax.ShapeDtypeStruct((m, d), jnp.float32),
        compiler_params=pltpu.CompilerParams(use_tc_tiling_on_sc=False),
        scratch_types=[
            pltpu.VMEM((wg,), jnp.int32),
            pltpu.VMEM((wg, d), jnp.float32),
            pltpu.SemaphoreType.DMA,
        ],
    )
    def k(table_hbm, idx_hbm, out_hbm, idx_v, rows_v, sem):
        wid = lax.axis_index("s") * 2 + lax.axis_index("c")
        base = wid * per_w

        def body(j, carry):
            b = base + j * wg
            pltpu.sync_copy(idx_hbm.at[pl.ds(b, wg)], idx_v)
            pltpu.async_copy(table_hbm.at[idx_v], rows_v, sem).wait()
            pltpu.sync_copy(rows_v, out_hbm.at[pl.ds(b, wg)])
            return carry

        lax.fori_loop(0, per_w // wg, body, 0)

    return k(table, idx)


def _pick_block(n, candidates=(2000, 1000, 500, 250, 100, 50, 25, 8, 4, 2, 1)):
    for c in candidates:
        if n % c == 0:
            return c
    return 1


def _embed_body(x_ref, w_ref, b_ref, ew_ref, o_ref):
    y = jnp.dot(x_ref[...], w_ref[...], preferred_element_type=jnp.float32)
    y = y + b_ref[...]
    col = lax.broadcasted_iota(jnp.int32, y.shape, 1)
    o_ref[...] = jnp.where(col == y.shape[1] - 1, ew_ref[...], y)


def _embed(elem_fea, w32, b32, elem_weights):
    n, emb = elem_fea.shape
    d = w32.shape[1]
    blk = _pick_block(n)
    return pl.pallas_call(
        _embed_body,
        grid=(n // blk,),
        in_specs=[
            pl.BlockSpec((blk, emb), lambda i: (i, 0)),
            pl.BlockSpec((emb, d), lambda i: (0, 0)),
            pl.BlockSpec((1, d), lambda i: (0, 0)),
            pl.BlockSpec((blk, 1), lambda i: (i, 0)),
        ],
        out_specs=pl.BlockSpec((blk, d), lambda i: (i, 0)),
        out_shape=jax.ShapeDtypeStruct((n, d), jnp.float32),
    )(elem_fea, w32, b32, elem_weights)


def _gm_body(fs_ref, fn_ref, nw_ref, w1a_ref, w1b_ref, b1_ref,
             w2g_ref, b2g_ref, w2m_ref, b2m_ref, e_ref, p16_ref,
             av_ref):
    h = jnp.dot(fs_ref[...], w1a_ref[...], preferred_element_type=jnp.float32)
    h = h + jnp.dot(fn_ref[...], w1b_ref[...], preferred_element_type=jnp.float32)
    h = jnp.maximum(h + b1_ref[...], 0.0)
    g = jnp.dot(h, w2g_ref[...], preferred_element_type=jnp.float32) + b2g_ref[...]
    a = nw_ref[...][:, :1] * jnp.exp(g)
    mg = jnp.dot(h, w2m_ref[...], preferred_element_type=jnp.float32) + b2m_ref[...]
    aexp = jnp.dot(a, e_ref[...], preferred_element_type=jnp.float32)
    a16 = jnp.dot(a, p16_ref[...], preferred_element_type=jnp.float32)
    av_ref[...] = jnp.concatenate([a16, aexp * mg], axis=1)


def _gate_msg(fs, fn, nw, w1a, w1b, b1, w2g, b2g, w2m, b2m, expand):
    r, din = fs.shape
    hid = w1a.shape[1]
    nh = w2g.shape[1]
    dm = w2m.shape[1]
    p16 = jnp.zeros((nh, 16), jnp.float32)
    for i in range(nh):
        p16 = p16.at[i, i].set(1.0)
    blk = _pick_block(r)
    full = lambda i: (0, 0)
    return pl.pallas_call(
        _gm_body,
        grid=(r // blk,),
        in_specs=[
            pl.BlockSpec((blk, din), lambda i: (i, 0)),
            pl.BlockSpec((blk, din), lambda i: (i, 0)),
            pl.BlockSpec((blk, 16), lambda i: (i, 0)),
            pl.BlockSpec((din, hid), full),
            pl.BlockSpec((din, hid), full),
            pl.BlockSpec((1, hid), full),
            pl.BlockSpec((hid, nh), full),
            pl.BlockSpec((1, nh), full),
            pl.BlockSpec((hid, dm), full),
            pl.BlockSpec((1, dm), full),
            pl.BlockSpec((nh, dm), full),
            pl.BlockSpec((nh, 16), full),
        ],
        out_specs=pl.BlockSpec((blk, 16 + dm), lambda i: (i, 0)),
        out_shape=jax.ShapeDtypeStruct((r, 16 + dm), jnp.float32),
    )(fs, fn, nw, w1a, w1b, b1, w2g, b2g, w2m, b2m, expand, p16)


_NV = 640
_EB = 128


@functools.partial(jax.jit, static_argnames=("nseg",))
def _sc_segsum(ev, idx, bounds, nseg):
    m, w = ev.shape
    nchunk_real = -(-nseg // _NV)
    rounds = -(-nchunk_real // _NUM_WORKERS)
    nchunk = rounds * _NUM_WORKERS
    npad = nchunk * _NV
    ncg = w // 16
    mesh = plsc.VectorSubcoreMesh(core_axis_name="c", subcore_axis_name="s")

    @functools.partial(
        pl.kernel, mesh=mesh,
        out_type=jax.ShapeDtypeStruct((npad, w), jnp.float32),
        compiler_params=pltpu.CompilerParams(use_tc_tiling_on_sc=False),
        scratch_types=[
            pltpu.VMEM((_NV + 8, w), jnp.float32),
            pltpu.VMEM((_EB, w), jnp.float32),
            pltpu.VMEM((_EB + 16,), jnp.int32),
            pltpu.VMEM((len(bounds),), jnp.int32),
        ],
    )
    def k(ev_hbm, idx_hbm, bnd_hbm, out_hbm, acc_v, ev_v, idx_v, bnd_v):
        wid = lax.axis_index("s") * 2 + lax.axis_index("c")
        pltpu.sync_copy(bnd_hbm, bnd_v)
        zeros16 = jnp.zeros((16,), jnp.float32)

        def round_body(r, carry):
            ck = r * _NUM_WORKERS + wid
            nb = ck * _NV
            bv = bnd_v[pl.ds(ck, 16)]
            e0 = bv[0]
            e1 = bv[1]

            def zrow(i, c):
                for cg in range(ncg):
                    acc_v[i, pl.ds(cg * 16, 16)] = zeros16
                return c
            lax.fori_loop(0, _NV + 8, zrow, 0)

            wb0 = e0 - (e0 & 7)
            nwin = (e1 - wb0 + _EB - 1) // _EB

            def win(j, c):
                s = wb0 + j * _EB
                bc = pl.multiple_of(jnp.minimum(s, m - _EB), 8)
                lo = s - bc
                pltpu.sync_copy(ev_hbm.at[pl.ds(bc, _EB)], ev_v)
                pltpu.sync_copy(idx_hbm.at[pl.ds(bc, _EB)], idx_v.at[pl.ds(0, _EB)])

                def edge(e, c2):
                    lr = idx_v[pl.ds(e, 16)][0] - nb
                    oob = (lr < 0) | (lr >= _NV) | (e < lo)
                    lr = jnp.where(oob, _NV, lr)
                    for cg in range(ncg):
                        v = ev_v[e, pl.ds(cg * 16, 16)]
                        plsc.addupdate(acc_v.at[lr, pl.ds(cg * 16, 16)], v)
                    return c2
                lax.fori_loop(0, _EB, edge, 0, unroll=8)
                return c
            lax.fori_loop(0, nwin, win, 0)
            pltpu.sync_copy(acc_v.at[pl.ds(0, _NV)], out_hbm.at[pl.ds(nb, _NV)])
            return carry

        lax.fori_loop(0, rounds, round_body, 0)

    return k(ev, idx, bounds)[:nseg]


def _seg_bounds(idx, nseg):
    nchunk_real = -(-nseg // _NV)
    rounds = -(-nchunk_real // _NUM_WORKERS)
    nchunk = rounds * _NUM_WORKERS
    nb = nchunk + 16
    starts = jnp.arange(nb, dtype=jnp.int32) * _NV
    return jnp.searchsorted(idx, starts, side="left").astype(jnp.int32)


def _stack_heads(heads, din, split):
    hid = heads[0]["gate"]["hidden"][0][0].shape[1]
    nh = len(heads)
    dmsg = heads[0]["msg"]["out"][0].shape[1]
    w1a_parts, w1b_parts, b1_parts = [], [], []
    for h in heads:
        for net in (h["gate"], h["msg"]):
            w1, b1 = net["hidden"][0]
            w1a_parts.append(w1[:split])
            w1b_parts.append(w1[split:] if w1.shape[0] > split
                             else jnp.zeros((split, hid), jnp.float32))
            b1_parts.append(b1)
    order = []
    for i in range(nh):
        order.append(2 * i)
    for i in range(nh):
        order.append(2 * i + 1)
    w1a = jnp.concatenate([w1a_parts[i] for i in order], axis=1)
    w1b = jnp.concatenate([w1b_parts[i] for i in order], axis=1)
    b1 = jnp.concatenate([b1_parts[i] for i in order], axis=0)[None, :]

    tot = 2 * nh * hid
    w2g = jnp.zeros((tot, nh), jnp.float32)
    b2g = jnp.zeros((1, nh), jnp.float32)
    w2m = jnp.zeros((tot, nh * dmsg), jnp.float32)
    b2m = jnp.zeros((1, nh * dmsg), jnp.float32)
    expand = jnp.zeros((nh, nh * dmsg), jnp.float32)
    for i, h in enumerate(heads):
        wg, bg = h["gate"]["out"]
        w2g = w2g.at[i * hid:(i + 1) * hid, i].set(wg[:, 0])
        b2g = b2g.at[0, i].set(bg[0])
        wm, bm = h["msg"]["out"]
        w2m = w2m.at[(nh + i) * hid:(nh + i + 1) * hid,
                     i * dmsg:(i + 1) * dmsg].set(wm)
        b2m = b2m.at[0, i * dmsg:(i + 1) * dmsg].set(bm)
        expand = expand.at[i, i * dmsg:(i + 1) * dmsg].set(1.0)
    return w1a, w1b, b1, w2g, b2g, w2m, b2m, expand


def _pool(av_seg, nh, dmsg):
    r = av_seg.shape[0]
    num = av_seg[:, 16:].reshape(r, nh, dmsg)
    den = av_seg[:, :nh, None]
    return jnp.mean(num / (den + 1e-10), axis=1)


def kernel(elem_weights, elem_fea, self_fea_idx, nbr_fea_idx, cry_elem_idx,
           V_window, rate, cycle, Vii, params):
    n = elem_fea.shape[0]
    m = self_fea_idx.shape[0]
    c = V_window.shape[0]

    w_emb, b_emb = params["emb"]
    d = w_emb.shape[1] + 1
    w32 = jnp.concatenate([w_emb, jnp.zeros((w_emb.shape[0], 1), jnp.float32)], axis=1)
    b32 = jnp.concatenate([b_emb, jnp.zeros((1,), jnp.float32)], axis=0)[None, :]

    fea = _embed(elem_fea, w32, b32, elem_weights)

    we16 = jnp.broadcast_to(elem_weights, (n, 16))
    nw = _sc_gather(we16, nbr_fea_idx)
    nh = len(params["graphs"][0])
    dmsg = params["graphs"][0][0]["msg"]["out"][0].shape[1]

    self_bounds = _seg_bounds(self_fea_idx, n)
    cry_bounds = _seg_bounds(cry_elem_idx, c)

    for heads in params["graphs"]:
        w1a, w1b, b1, w2g, b2g, w2m, b2m, expand = _stack_heads(heads, 2 * d, d)
        fs = _sc_gather(fea, self_fea_idx)
        fn = _sc_gather(fea, nbr_fea_idx)
        av = _gate_msg(fs, fn, nw, w1a, w1b, b1, w2g, b2g, w2m, b2m, expand)
        av_seg = _sc_segsum(av, self_fea_idx, self_bounds, n)
        fea = fea + _pool(av_seg, nh, dmsg)

    w1a, w1b, b1, w2g, b2g, w2m, b2m, expand = _stack_heads(params["cry"], d, d)
    av = _gate_msg(fea, fea, we16, w1a, w1b, b1,
                   w2g, b2g, w2m, b2m, expand)
    av_seg = _sc_segsum(av, cry_elem_idx, cry_bounds, c)
    cry_fea = _pool(av_seg, nh, dmsg)
    return (cry_fea, fea)


# speedup vs baseline: 1.0537x; 1.0537x over previous
import functools

import jax
import jax.numpy as jnp
from jax import lax
from jax.experimental import pallas as pl
from jax.experimental.pallas import tpu as pltpu
from jax.experimental.pallas import tpu_sc as plsc

_NUM_WORKERS = 32


@functools.partial(jax.jit, static_argnames=("wg",))
def _sc_gather(table, idx, wg=1000):
    m = idx.shape[0]
    d = table.shape[1]
    per_w = m // _NUM_WORKERS
    mesh = plsc.VectorSubcoreMesh(core_axis_name="c", subcore_axis_name="s")

    @functools.partial(
        pl.kernel, mesh=mesh,
        out_type=jax.ShapeDtypeStruct((m, d), jnp.float32),
        compiler_params=pltpu.CompilerParams(use_tc_tiling_on_sc=False),
        scratch_types=[
            pltpu.VMEM((wg,), jnp.int32),
            pltpu.VMEM((wg, d), jnp.float32),
            pltpu.SemaphoreType.DMA,
        ],
    )
    def k(table_hbm, idx_hbm, out_hbm, idx_v, rows_v, sem):
        wid = lax.axis_index("s") * 2 + lax.axis_index("c")
        base = wid * per_w

        def body(j, carry):
            b = base + j * wg
            pltpu.sync_copy(idx_hbm.at[pl.ds(b, wg)], idx_v)
            pltpu.async_copy(table_hbm.at[idx_v], rows_v, sem).wait()
            pltpu.sync_copy(rows_v, out_hbm.at[pl.ds(b, wg)])
            return carry

        lax.fori_loop(0, per_w // wg, body, 0)

    return k(table, idx)


def _pick_block(n, candidates=(2000, 1000, 500, 250, 100, 50, 25, 8, 4, 2, 1)):
    for c in candidates:
        if n % c == 0:
            return c
    return 1


def _embed_body(x_ref, w_ref, b_ref, ew_ref, o_ref):
    y = jnp.dot(x_ref[...], w_ref[...], preferred_element_type=jnp.float32)
    y = y + b_ref[...]
    col = lax.broadcasted_iota(jnp.int32, y.shape, 1)
    o_ref[...] = jnp.where(col == y.shape[1] - 1, ew_ref[...], y)


def _embed(elem_fea, w32, b32, elem_weights):
    n, emb = elem_fea.shape
    d = w32.shape[1]
    blk = _pick_block(n)
    return pl.pallas_call(
        _embed_body,
        grid=(n // blk,),
        in_specs=[
            pl.BlockSpec((blk, emb), lambda i: (i, 0)),
            pl.BlockSpec((emb, d), lambda i: (0, 0)),
            pl.BlockSpec((1, d), lambda i: (0, 0)),
            pl.BlockSpec((blk, 1), lambda i: (i, 0)),
        ],
        out_specs=pl.BlockSpec((blk, d), lambda i: (i, 0)),
        out_shape=jax.ShapeDtypeStruct((n, d), jnp.float32),
    )(elem_fea, w32, b32, elem_weights)


def _gm_body(fs_ref, fn_ref, nw_ref, w1a_ref, b1_ref,
             w2g_ref, b2g_ref, w2m_ref, b2m_ref, e_ref, p16_ref,
             av_ref):
    pair = jnp.concatenate([fs_ref[...], fn_ref[...]], axis=1)
    h = jnp.dot(pair, w1a_ref[...], preferred_element_type=jnp.float32)
    h = jnp.maximum(h + b1_ref[...], 0.0)
    g = jnp.dot(h, w2g_ref[...], preferred_element_type=jnp.float32) + b2g_ref[...]
    a = nw_ref[...][:, :1] * jnp.exp(g)
    mg = jnp.dot(h, w2m_ref[...], preferred_element_type=jnp.float32) + b2m_ref[...]
    aexp = jnp.dot(a, e_ref[...], preferred_element_type=jnp.float32)
    a16 = jnp.dot(a, p16_ref[...], preferred_element_type=jnp.float32)
    av_ref[...] = jnp.concatenate([a16, aexp * mg], axis=1)


def _gate_msg(fs, fn, nw, w1a, w1b, b1, w2g, b2g, w2m, b2m, expand):
    r, din = fs.shape
    hid = w1a.shape[1]
    nh = w2g.shape[1]
    dm = w2m.shape[1]
    p16 = jnp.zeros((nh, 16), jnp.float32)
    for i in range(nh):
        p16 = p16.at[i, i].set(1.0)
    w1cat = jnp.concatenate([w1a, w1b], axis=0)
    blk = _pick_block(r)
    full = lambda i: (0, 0)
    return pl.pallas_call(
        _gm_body,
        grid=(r // blk,),
        in_specs=[
            pl.BlockSpec((blk, din), lambda i: (i, 0)),
            pl.BlockSpec((blk, din), lambda i: (i, 0)),
            pl.BlockSpec((blk, 16), lambda i: (i, 0)),
            pl.BlockSpec((2 * din, hid), full),
            pl.BlockSpec((1, hid), full),
            pl.BlockSpec((hid, nh), full),
            pl.BlockSpec((1, nh), full),
            pl.BlockSpec((hid, dm), full),
            pl.BlockSpec((1, dm), full),
            pl.BlockSpec((nh, dm), full),
            pl.BlockSpec((nh, 16), full),
        ],
        out_specs=pl.BlockSpec((blk, 16 + dm), lambda i: (i, 0)),
        out_shape=jax.ShapeDtypeStruct((r, 16 + dm), jnp.float32),
    )(fs, fn, nw, w1cat, b1, w2g, b2g, w2m, b2m, expand, p16)


_NV = 640
_EB = 256


@functools.partial(jax.jit, static_argnames=("nseg",))
def _sc_segsum(ev, idx, bounds, nseg):
    m, w = ev.shape
    nchunk_real = -(-nseg // _NV)
    rounds = -(-nchunk_real // _NUM_WORKERS)
    nchunk = rounds * _NUM_WORKERS
    npad = nchunk * _NV
    ncg = w // 16
    mesh = plsc.VectorSubcoreMesh(core_axis_name="c", subcore_axis_name="s")

    @functools.partial(
        pl.kernel, mesh=mesh,
        out_type=jax.ShapeDtypeStruct((npad, w), jnp.float32),
        compiler_params=pltpu.CompilerParams(use_tc_tiling_on_sc=False),
        scratch_types=[
            pltpu.VMEM((_NV + 8, w), jnp.float32),
            pltpu.VMEM((_EB, w), jnp.float32),
            pltpu.VMEM((_EB + 16,), jnp.int32),
            pltpu.VMEM((len(bounds),), jnp.int32),
        ],
    )
    def k(ev_hbm, idx_hbm, bnd_hbm, out_hbm, acc_v, ev_v, idx_v, bnd_v):
        wid = lax.axis_index("s") * 2 + lax.axis_index("c")
        pltpu.sync_copy(bnd_hbm, bnd_v)
        zeros16 = jnp.zeros((16,), jnp.float32)

        def round_body(r, carry):
            ck = r * _NUM_WORKERS + wid
            nb = ck * _NV
            bv = bnd_v[pl.ds(ck, 16)]
            e0 = bv[0]
            e1 = bv[1]

            def zrow(i, c):
                for cg in range(ncg):
                    acc_v[i, pl.ds(cg * 16, 16)] = zeros16
                return c
            lax.fori_loop(0, _NV + 8, zrow, 0)

            wb0 = e0 - (e0 & 7)
            nwin = (e1 - wb0 + _EB - 1) // _EB

            def win(j, c):
                s = wb0 + j * _EB
                bc = pl.multiple_of(jnp.minimum(s, m - _EB), 8)
                lo = s - bc
                pltpu.sync_copy(ev_hbm.at[pl.ds(bc, _EB)], ev_v)
                pltpu.sync_copy(idx_hbm.at[pl.ds(bc, _EB)], idx_v.at[pl.ds(0, _EB)])

                def edge(e, c2):
                    lr = idx_v[pl.ds(e, 16)][0] - nb
                    oob = (lr < 0) | (lr >= _NV) | (e < lo)
                    lr = jnp.where(oob, _NV, lr)
                    for cg in range(ncg):
                        v = ev_v[e, pl.ds(cg * 16, 16)]
                        plsc.addupdate(acc_v.at[lr, pl.ds(cg * 16, 16)], v)
                    return c2
                lax.fori_loop(0, _EB, edge, 0, unroll=8)
                return c
            lax.fori_loop(0, nwin, win, 0)
            pltpu.sync_copy(acc_v.at[pl.ds(0, _NV)], out_hbm.at[pl.ds(nb, _NV)])
            return carry

        lax.fori_loop(0, rounds, round_body, 0)

    return k(ev, idx, bounds)[:nseg]


def _seg_bounds(idx, nseg):
    nchunk_real = -(-nseg // _NV)
    rounds = -(-nchunk_real // _NUM_WORKERS)
    nchunk = rounds * _NUM_WORKERS
    nb = nchunk + 16
    starts = jnp.arange(nb, dtype=jnp.int32) * _NV
    return jnp.searchsorted(idx, starts, side="left").astype(jnp.int32)


def _stack_heads(heads, din, split):
    hid = heads[0]["gate"]["hidden"][0][0].shape[1]
    nh = len(heads)
    dmsg = heads[0]["msg"]["out"][0].shape[1]
    w1a_parts, w1b_parts, b1_parts = [], [], []
    for h in heads:
        for net in (h["gate"], h["msg"]):
            w1, b1 = net["hidden"][0]
            w1a_parts.append(w1[:split])
            w1b_parts.append(w1[split:] if w1.shape[0] > split
                             else jnp.zeros((split, hid), jnp.float32))
            b1_parts.append(b1)
    order = []
    for i in range(nh):
        order.append(2 * i)
    for i in range(nh):
        order.append(2 * i + 1)
    w1a = jnp.concatenate([w1a_parts[i] for i in order], axis=1)
    w1b = jnp.concatenate([w1b_parts[i] for i in order], axis=1)
    b1 = jnp.concatenate([b1_parts[i] for i in order], axis=0)[None, :]

    tot = 2 * nh * hid
    w2g = jnp.zeros((tot, nh), jnp.float32)
    b2g = jnp.zeros((1, nh), jnp.float32)
    w2m = jnp.zeros((tot, nh * dmsg), jnp.float32)
    b2m = jnp.zeros((1, nh * dmsg), jnp.float32)
    expand = jnp.zeros((nh, nh * dmsg), jnp.float32)
    for i, h in enumerate(heads):
        wg, bg = h["gate"]["out"]
        w2g = w2g.at[i * hid:(i + 1) * hid, i].set(wg[:, 0])
        b2g = b2g.at[0, i].set(bg[0])
        wm, bm = h["msg"]["out"]
        w2m = w2m.at[(nh + i) * hid:(nh + i + 1) * hid,
                     i * dmsg:(i + 1) * dmsg].set(wm)
        b2m = b2m.at[0, i * dmsg:(i + 1) * dmsg].set(bm)
        expand = expand.at[i, i * dmsg:(i + 1) * dmsg].set(1.0)
    return w1a, w1b, b1, w2g, b2g, w2m, b2m, expand


def _pool(av_seg, nh, dmsg):
    r = av_seg.shape[0]
    num = av_seg[:, 16:].reshape(r, nh, dmsg)
    den = av_seg[:, :nh, None]
    return jnp.mean(num / (den + 1e-10), axis=1)


def kernel(elem_weights, elem_fea, self_fea_idx, nbr_fea_idx, cry_elem_idx,
           V_window, rate, cycle, Vii, params):
    n = elem_fea.shape[0]
    m = self_fea_idx.shape[0]
    c = V_window.shape[0]

    w_emb, b_emb = params["emb"]
    d = w_emb.shape[1] + 1
    w32 = jnp.concatenate([w_emb, jnp.zeros((w_emb.shape[0], 1), jnp.float32)], axis=1)
    b32 = jnp.concatenate([b_emb, jnp.zeros((1,), jnp.float32)], axis=0)[None, :]

    fea = _embed(elem_fea, w32, b32, elem_weights)

    we16 = jnp.broadcast_to(elem_weights, (n, 16))
    nw = _sc_gather(we16, nbr_fea_idx)
    nh = len(params["graphs"][0])
    dmsg = params["graphs"][0][0]["msg"]["out"][0].shape[1]

    self_bounds = _seg_bounds(self_fea_idx, n)
    cry_bounds = _seg_bounds(cry_elem_idx, c)

    for heads in params["graphs"]:
        w1a, w1b, b1, w2g, b2g, w2m, b2m, expand = _stack_heads(heads, 2 * d, d)
        fs = _sc_gather(fea, self_fea_idx)
        fn = _sc_gather(fea, nbr_fea_idx)
        av = _gate_msg(fs, fn, nw, w1a, w1b, b1, w2g, b2g, w2m, b2m, expand)
        av_seg = _sc_segsum(av, self_fea_idx, self_bounds, n)
        fea = fea + _pool(av_seg, nh, dmsg)

    w1a, w1b, b1, w2g, b2g, w2m, b2m, expand = _stack_heads(params["cry"], d, d)
    av = _gate_msg(fea, fea, we16, w1a, w1b, b1,
                   w2g, b2g, w2m, b2m, expand)
    av_seg = _sc_segsum(av, cry_elem_idx, cry_bounds, c)
    cry_fea = _pool(av_seg, nh, dmsg)
    return (cry_fea, fea)
